# baseline (speedup 1.0000x reference)
import functools

import jax
import jax.numpy as jnp
from jax import lax
from jax.experimental import layout as jax_layout
from jax.experimental import pallas as pl
from jax.experimental.pallas import tpu as pltpu
from jax.experimental.pallas import tpu_sc as plsc

R = 128
V = 100000
NC = 2
NS = 16
TILE = 128

TC_CHUNK = 7808
TC_GRID = 12
TC_COLS = TC_CHUNK * TC_GRID
SC_COL0 = TC_COLS
SC_V = V - SC_COL0
SC_W1 = 3200
SC_W2 = 3072
SC_TAIL0 = SC_W1 + SC_W2
SC_WTAIL = SC_V - SC_TAIL0
IBIG = 2 ** 30

def _tc_argmax_body(x_ref, ov_ref, oi_ref, cm, ci):
    pid = pl.program_id(0)
    x = x_ref[...]
    m = jnp.max(x, axis=1, keepdims=True)
    base = pid * TC_CHUNK
    iota = lax.broadcasted_iota(jnp.int32, x.shape, 1) + base
    cand = jnp.where(x == m, iota, jnp.int32(IBIG))
    mi = jnp.min(cand, axis=1, keepdims=True)

    @pl.when(pid == 0)
    def _():
        cm[...] = m
        ci[...] = mi

    @pl.when(pid > 0)
    def _():
        better = m > cm[...]
        ci[...] = jnp.where(better, mi, ci[...])
        cm[...] = jnp.where(better, m, cm[...])

    @pl.when(pid == TC_GRID - 1)
    def _():
        ov_ref[...] = cm[...]
        oi_ref[...] = ci[...]


_tc_argmax = pl.pallas_call(
    _tc_argmax_body,
    grid=(TC_GRID,),
    in_specs=[pl.BlockSpec((R, TC_CHUNK), lambda i: (0, i))],
    out_specs=[pl.BlockSpec((R, 1), lambda i: (0, 0)),
               pl.BlockSpec((R, 1), lambda i: (0, 0))],
    out_shape=[jax.ShapeDtypeStruct((R, 1), jnp.float32),
               jax.ShapeDtypeStruct((R, 1), jnp.int32)],
    scratch_shapes=[pltpu.VMEM((R, 1), jnp.float32),
                    pltpu.VMEM((R, 1), jnp.int32)],
)

_mesh = plsc.VectorSubcoreMesh(core_axis_name="c", subcore_axis_name="s")


@functools.partial(
    pl.kernel,
    out_type=(jax.ShapeDtypeStruct((16, 16), jnp.float32),
              jax.ShapeDtypeStruct((16, 16), jnp.int32)),
    mesh=_mesh,
    scratch_types=[
        pltpu.VMEM((8, SC_W1), jnp.float32),
        pltpu.VMEM((8, SC_W2), jnp.float32),
        pltpu.VMEM((8, SC_WTAIL), jnp.float32),
        pltpu.VMEM((16,), jnp.float32),
        pltpu.VMEM((16,), jnp.int32),
        pltpu.SemaphoreType.DMA,
        pltpu.SemaphoreType.DMA,
        pltpu.SemaphoreType.DMA,
    ],
)
def _sc_shard(x_hbm, ov_hbm, oi_hbm, buf1, buf2, buft, res_f, res_i,
              sem1, sem2, sem3):
    cid = lax.axis_index("c")
    sid = lax.axis_index("s")
    blk = cid * 8 + sid // 2
    row0 = pl.multiple_of(blk * 8, 8)
    lane = lax.iota(jnp.int32, 16)

    src1 = x_hbm.at[pl.ds(row0, 8), pl.ds(0, SC_W1)]
    src2 = x_hbm.at[pl.ds(row0, 8), pl.ds(SC_W1, SC_W2)]
    srct = x_hbm.at[pl.ds(row0, 8), pl.ds(SC_TAIL0, SC_WTAIL)]
    pltpu.async_copy(src1, buf1, sem1)
    pltpu.async_copy(src2, buf2, sem2)
    pltpu.async_copy(srct, buft, sem3)

    vm = [jnp.full((16,), -jnp.inf, jnp.float32) for _ in range(8)]
    vi = [jnp.zeros((16,), jnp.int32) for _ in range(8)]

    def scan_chunk(buf, w, colbase, vm, vi):
        def body(s, carry):
            cvm = list(carry[0])
            cvi = list(carry[1])
            c0 = s * 32
            colv = (colbase + c0) + lane
            for u in range(2):
                idx = colv + u * 16
                for wr in range(8):
                    v = buf[wr, pl.ds(c0 + u * 16, 16)]
                    pred = v > cvm[wr]
                    cvm[wr] = jnp.where(pred, v, cvm[wr])
                    cvi[wr] = jnp.where(pred, idx, cvi[wr])
            return tuple(cvm), tuple(cvi)

        out = lax.fori_loop(0, w // 32, body, (tuple(vm), tuple(vi)))
        return list(out[0]), list(out[1])

    pltpu.make_async_copy(src1, buf1, sem1).wait()
    vm, vi = scan_chunk(buf1, SC_W1, jnp.int32(SC_COL0), vm, vi)
    pltpu.make_async_copy(src2, buf2, sem2).wait()
    vm, vi = scan_chunk(buf2, SC_W2, jnp.int32(SC_COL0 + SC_W1), vm, vi)
    pltpu.make_async_copy(srct, buft, sem3).wait()

    tail_col = jnp.int32(SC_COL0 + SC_TAIL0) + lane
    for u in range(2):
        idx = tail_col + u * 16
        for wr in range(8):
            v = buft[wr, pl.ds(u * 16, 16)]
            pred = v > vm[wr]
            vm[wr] = jnp.where(pred, v, vm[wr])
            vi[wr] = jnp.where(pred, idx, vi[wr])

    rf = jnp.full((16,), -jnp.inf, jnp.float32)
    ri = jnp.zeros((16,), jnp.int32)
    for wr in range(8):
        m = vm[wr][0]
        mi = vi[wr][0]
        for l in range(1, 16):
            v = vm[wr][l]
            i = vi[wr][l]
            better = (v > m) | ((v == m) & (i < mi))
            m = jnp.where(better, v, m)
            mi = jnp.where(better, i, mi)
        rf = jnp.where(lane == wr, m, rf)
        ri = jnp.where(lane == wr, mi, ri)

    res_f[...] = rf
    res_i[...] = ri
    pltpu.sync_copy(res_f, ov_hbm.at[blk])
    pltpu.sync_copy(res_i, oi_hbm.at[blk])

def _merge_body(tv_ref, ti_ref, sv_ref, si_ref, o_ref):
    better = sv_ref[...] > tv_ref[...]
    o_ref[...] = jnp.where(better, si_ref[...], ti_ref[...])


_merge = pl.pallas_call(
    _merge_body,
    out_shape=jax.ShapeDtypeStruct((R, 1), jnp.int32),
)

def _kernel_impl(m_logits):
    tv, ti = _tc_argmax(m_logits)
    sv, si = _sc_shard(m_logits[:, SC_COL0:])
    sv = sv[:, :8].reshape(R, 1)
    si = si[:, :8].reshape(R, 1)
    token = _merge(tv, ti, sv, si)
    return token.astype(jnp.int64)


_plain_jit = jax.jit(_kernel_impl)
_jit_cache = {}


def kernel(m_logits):
    try:
        sharding = m_logits.sharding
    except AttributeError:
        return _plain_jit(m_logits)
    fn = _jit_cache.get(sharding)
    if fn is None:
        fmt = jax_layout.Format(
            jax_layout.Layout(major_to_minor=(0, 1)), sharding)
        fn = jax.jit(_kernel_impl, in_shardings=(fmt,))
        _jit_cache[sharding] = fn
    return fn(m_logits)


# speedup vs baseline: 1.8660x; 1.8660x over previous
import functools

import jax
import jax.numpy as jnp
from jax import lax
from jax.experimental import pallas as pl
from jax.experimental.pallas import tpu as pltpu
from jax.experimental.pallas import tpu_sc as plsc

R = 128
V = 100000
NW = 32

TC_CHUNK = 5864
TC_GRID = 16
TC_V = TC_CHUNK * TC_GRID
SC_V0 = TC_V
SC_STRIPE = 192
SC_TAIL0 = SC_V0 + NW * SC_STRIPE
SC_WTAIL = V - SC_TAIL0
IBIG = 2 ** 30

def _tc_argmax_body(x_ref, ov_ref, oi_ref, cm, ci):
    pid = pl.program_id(0)
    x = x_ref[...]
    m = jnp.max(x, axis=0, keepdims=True)
    base = pid * TC_CHUNK
    iota = lax.broadcasted_iota(jnp.int32, x.shape, 0) + base
    cand = jnp.where(x == m, iota, jnp.int32(IBIG))
    mi = jnp.min(cand, axis=0, keepdims=True)

    @pl.when(pid == 0)
    def _():
        cm[...] = m
        ci[...] = mi

    @pl.when(pid > 0)
    def _():
        better = m > cm[...]
        ci[...] = jnp.where(better, mi, ci[...])
        cm[...] = jnp.where(better, m, cm[...])

    @pl.when(pid == TC_GRID - 1)
    def _():
        ov_ref[...] = cm[...]
        oi_ref[...] = ci[...]


_tc_argmax = pl.pallas_call(
    _tc_argmax_body,
    grid=(TC_GRID,),
    in_specs=[pl.BlockSpec((TC_CHUNK, R), lambda i: (i, 0))],
    out_specs=[pl.BlockSpec((1, R), lambda i: (0, 0)),
               pl.BlockSpec((1, R), lambda i: (0, 0))],
    out_shape=[jax.ShapeDtypeStruct((1, R), jnp.float32),
               jax.ShapeDtypeStruct((1, R), jnp.int32)],
    scratch_shapes=[pltpu.VMEM((1, R), jnp.float32),
                    pltpu.VMEM((1, R), jnp.int32)],
)

_mesh = plsc.VectorSubcoreMesh(core_axis_name="c", subcore_axis_name="s")


@functools.partial(
    pl.kernel,
    out_type=(jax.ShapeDtypeStruct((NW, R), jnp.float32),
              jax.ShapeDtypeStruct((NW, R), jnp.int32)),
    mesh=_mesh,
    scratch_types=[
        pltpu.VMEM((SC_STRIPE, R), jnp.float32),
        pltpu.VMEM((SC_WTAIL, R), jnp.float32),
        pltpu.VMEM((R,), jnp.float32),
        pltpu.VMEM((R,), jnp.int32),
        pltpu.SemaphoreType.DMA,
        pltpu.SemaphoreType.DMA,
    ],
)
def _sc_shard(x_hbm, ov_hbm, oi_hbm, buf, buft, res_f, res_i, sem1, sem2):
    cid = lax.axis_index("c")
    sid = lax.axis_index("s")
    wid = cid * 16 + sid
    v0 = pl.multiple_of(SC_V0 + wid * SC_STRIPE, 8)

    src = x_hbm.at[pl.ds(v0, SC_STRIPE), :]
    srct = x_hbm.at[pl.ds(SC_TAIL0, SC_WTAIL), :]
    pltpu.async_copy(src, buf, sem1)
    pltpu.async_copy(srct, buft, sem2)

    vm0 = tuple(jnp.full((16,), -jnp.inf, jnp.float32) for _ in range(8))
    vi0 = tuple(jnp.zeros((16,), jnp.int32) for _ in range(8))

    def scan(bufref, n, vbase, carry):
        def body(s, c):
            cvm = list(c[0])
            cvi = list(c[1])
            idx = jnp.full((16,), vbase + s, jnp.int32)
            for g in range(8):
                v = bufref[s, pl.ds(g * 16, 16)]
                pred = v > cvm[g]
                cvm[g] = jnp.where(pred, v, cvm[g])
                cvi[g] = jnp.where(pred, idx, cvi[g])
            return tuple(cvm), tuple(cvi)

        return lax.fori_loop(0, n, body, carry)

    pltpu.make_async_copy(src, buf, sem1).wait()
    carry = scan(buf, SC_STRIPE, v0, (vm0, vi0))
    pltpu.make_async_copy(srct, buft, sem2).wait()
    vm, vi = scan(buft, SC_WTAIL, jnp.int32(SC_TAIL0), carry)

    for g in range(8):
        res_f[pl.ds(g * 16, 16)] = vm[g]
        res_i[pl.ds(g * 16, 16)] = vi[g]
    pltpu.sync_copy(res_f, ov_hbm.at[wid])
    pltpu.sync_copy(res_i, oi_hbm.at[wid])

def _merge_body(tv_ref, ti_ref, sv_ref, si_ref, o_ref):
    sv = sv_ref[...]
    m = jnp.max(sv, axis=0, keepdims=True)
    cand = jnp.where(sv == m, si_ref[...], jnp.int32(IBIG))
    smi = jnp.min(cand, axis=0, keepdims=True)
    better = m > tv_ref[...]
    o_ref[...] = jnp.where(better, smi, ti_ref[...])


_merge = pl.pallas_call(
    _merge_body,
    out_shape=jax.ShapeDtypeStruct((1, R), jnp.int32),
)

@jax.jit
def kernel(m_logits):
    x_t = m_logits.T
    tv, ti = _tc_argmax(x_t)
    sv, si = _sc_shard(x_t)
    token = _merge(tv, ti, sv, si)
    return token.reshape(R, 1).astype(jnp.int64)


# speedup vs baseline: 2.2431x; 1.2021x over previous
import functools

import jax
import jax.numpy as jnp
from jax import lax
from jax.experimental import pallas as pl
from jax.experimental.pallas import tpu as pltpu
from jax.experimental.pallas import tpu_sc as plsc

R = 128
V = 100000
NW = 32

TC_CHUNK = 4816
TC_GRID = 8
TC_V = TC_CHUNK * TC_GRID
SC_V0 = TC_V
SC_STRIPE = 1920
SC_CW = 384
SC_NCH = SC_STRIPE // SC_CW
SC_TAIL0 = SC_V0 + NW * SC_STRIPE
SC_WTAIL = V - SC_TAIL0
IBIG = 2 ** 30

def _tc_argmax_body(x_ref, ov_ref, oi_ref, cm, ci):
    pid = pl.program_id(0)
    x = x_ref[...]
    m = jnp.max(x, axis=0, keepdims=True)
    base = pid * TC_CHUNK
    iota = lax.broadcasted_iota(jnp.int32, x.shape, 0) + base
    cand = jnp.where(x == m, iota, jnp.int32(IBIG))
    mi = jnp.min(cand, axis=0, keepdims=True)

    @pl.when(pid == 0)
    def _():
        cm[...] = m
        ci[...] = mi

    @pl.when(pid > 0)
    def _():
        better = m > cm[...]
        ci[...] = jnp.where(better, mi, ci[...])
        cm[...] = jnp.where(better, m, cm[...])

    @pl.when(pid == TC_GRID - 1)
    def _():
        ov_ref[...] = cm[...]
        oi_ref[...] = ci[...]


_tc_argmax = pl.pallas_call(
    _tc_argmax_body,
    grid=(TC_GRID,),
    in_specs=[pl.BlockSpec((TC_CHUNK, R), lambda i: (i, 0))],
    out_specs=[pl.BlockSpec((1, R), lambda i: (0, 0)),
               pl.BlockSpec((1, R), lambda i: (0, 0))],
    out_shape=[jax.ShapeDtypeStruct((1, R), jnp.float32),
               jax.ShapeDtypeStruct((1, R), jnp.int32)],
    scratch_shapes=[pltpu.VMEM((1, R), jnp.float32),
                    pltpu.VMEM((1, R), jnp.int32)],
)

_mesh = plsc.VectorSubcoreMesh(core_axis_name="c", subcore_axis_name="s")


@functools.partial(
    pl.kernel,
    out_type=(jax.ShapeDtypeStruct((NW, R), jnp.float32),
              jax.ShapeDtypeStruct((NW, R), jnp.int32)),
    mesh=_mesh,
    scratch_types=[
        pltpu.VMEM((SC_CW, R), jnp.float32),
        pltpu.VMEM((SC_CW, R), jnp.float32),
        pltpu.VMEM((SC_WTAIL, R), jnp.float32),
        pltpu.VMEM((R,), jnp.float32),
        pltpu.VMEM((R,), jnp.int32),
        pltpu.SemaphoreType.DMA,
        pltpu.SemaphoreType.DMA,
        pltpu.SemaphoreType.DMA,
    ],
)
def _sc_shard(x_hbm, ov_hbm, oi_hbm, buf0, buf1, buft, res_f, res_i,
              sem0, sem1, sem2):
    cid = lax.axis_index("c")
    sid = lax.axis_index("s")
    wid = cid * 16 + sid
    v0 = pl.multiple_of(SC_V0 + wid * SC_STRIPE, 8)
    bufs = (buf0, buf1)
    sems = (sem0, sem1)

    def chunk_src(j):
        return x_hbm.at[pl.ds(pl.multiple_of(v0 + j * SC_CW, 8), SC_CW), :]

    srct = x_hbm.at[pl.ds(SC_TAIL0, SC_WTAIL), :]
    pltpu.async_copy(chunk_src(0), buf0, sem0)
    pltpu.async_copy(chunk_src(1), buf1, sem1)
    pltpu.async_copy(srct, buft, sem2)

    carry = (tuple(jnp.full((16,), -jnp.inf, jnp.float32) for _ in range(8)),
             tuple(jnp.zeros((16,), jnp.int32) for _ in range(8)))

    def scan(bufref, n, vbase, carry):
        def body(s, c):
            cvm = list(c[0])
            cvi = list(c[1])
            idx = jnp.full((16,), vbase + s, jnp.int32)
            for g in range(8):
                v = bufref[s, pl.ds(g * 16, 16)]
                pred = v > cvm[g]
                cvm[g] = jnp.where(pred, v, cvm[g])
                cvi[g] = jnp.where(pred, idx, cvi[g])
            return tuple(cvm), tuple(cvi)

        return lax.fori_loop(0, n, body, carry)

    for j in range(SC_NCH):
        b = j % 2
        pltpu.make_async_copy(chunk_src(j), bufs[b], sems[b]).wait()
        carry = scan(bufs[b], SC_CW, v0 + j * SC_CW, carry)
        if j + 2 < SC_NCH:
            pltpu.async_copy(chunk_src(j + 2), bufs[b], sems[b])

    pltpu.make_async_copy(srct, buft, sem2).wait()
    vm, vi = scan(buft, SC_WTAIL, jnp.int32(SC_TAIL0), carry)

    for g in range(8):
        res_f[pl.ds(g * 16, 16)] = vm[g]
        res_i[pl.ds(g * 16, 16)] = vi[g]
    pltpu.sync_copy(res_f, ov_hbm.at[wid])
    pltpu.sync_copy(res_i, oi_hbm.at[wid])

def _merge_body(tv_ref, ti_ref, sv_ref, si_ref, o_ref):
    sv = sv_ref[...]
    m = jnp.max(sv, axis=0, keepdims=True)
    cand = jnp.where(sv == m, si_ref[...], jnp.int32(IBIG))
    smi = jnp.min(cand, axis=0, keepdims=True)
    better = m > tv_ref[...]
    o_ref[...] = jnp.where(better, smi, ti_ref[...])


_merge = pl.pallas_call(
    _merge_body,
    out_shape=jax.ShapeDtypeStruct((1, R), jnp.int32),
)

@jax.jit
def kernel(m_logits):
    x_t = m_logits.T
    tv, ti = _tc_argmax(x_t)
    sv, si = _sc_shard(x_t)
    token = _merge(tv, ti, sv, si)
    return token.reshape(R, 1).astype(jnp.int64)


# speedup vs baseline: 2.3226x; 1.0354x over previous
import functools

import jax
import jax.numpy as jnp
from jax import lax
from jax.experimental import pallas as pl
from jax.experimental.pallas import tpu as pltpu
from jax.experimental.pallas import tpu_sc as plsc

R = 128
V = 100000
NW = 32

TC_CHUNK = 6096
TC_GRID = 8
TC_V = TC_CHUNK * TC_GRID
SC_V0 = TC_V
SC_STRIPE = 1600
SC_CW = 320
SC_NCH = SC_STRIPE // SC_CW
SC_TAIL0 = SC_V0 + NW * SC_STRIPE
SC_WTAIL = V - SC_TAIL0
IBIG = 2 ** 30

def _tc_argmax_body(x_ref, ov_ref, oi_ref, cm, ci):
    pid = pl.program_id(0)
    x = x_ref[...]
    m = jnp.max(x, axis=0, keepdims=True)
    base = pid * TC_CHUNK
    iota = lax.broadcasted_iota(jnp.int32, x.shape, 0) + base
    cand = jnp.where(x == m, iota, jnp.int32(IBIG))
    mi = jnp.min(cand, axis=0, keepdims=True)

    @pl.when(pid == 0)
    def _():
        cm[...] = m
        ci[...] = mi

    @pl.when(pid > 0)
    def _():
        better = m > cm[...]
        ci[...] = jnp.where(better, mi, ci[...])
        cm[...] = jnp.where(better, m, cm[...])

    @pl.when(pid == TC_GRID - 1)
    def _():
        ov_ref[...] = cm[...]
        oi_ref[...] = ci[...]


_tc_argmax = pl.pallas_call(
    _tc_argmax_body,
    grid=(TC_GRID,),
    in_specs=[pl.BlockSpec((TC_CHUNK, R), lambda i: (i, 0))],
    out_specs=[pl.BlockSpec((1, R), lambda i: (0, 0)),
               pl.BlockSpec((1, R), lambda i: (0, 0))],
    out_shape=[jax.ShapeDtypeStruct((1, R), jnp.float32),
               jax.ShapeDtypeStruct((1, R), jnp.int32)],
    scratch_shapes=[pltpu.VMEM((1, R), jnp.float32),
                    pltpu.VMEM((1, R), jnp.int32)],
)

_mesh = plsc.VectorSubcoreMesh(core_axis_name="c", subcore_axis_name="s")


@functools.partial(
    pl.kernel,
    out_type=(jax.ShapeDtypeStruct((NW, R), jnp.float32),
              jax.ShapeDtypeStruct((NW, R), jnp.int32)),
    mesh=_mesh,
    scratch_types=[
        pltpu.VMEM((SC_CW, R), jnp.float32),
        pltpu.VMEM((SC_CW, R), jnp.float32),
        pltpu.VMEM((SC_WTAIL, R), jnp.float32),
        pltpu.VMEM((R,), jnp.float32),
        pltpu.VMEM((R,), jnp.int32),
        pltpu.SemaphoreType.DMA,
        pltpu.SemaphoreType.DMA,
        pltpu.SemaphoreType.DMA,
    ],
)
def _sc_shard(x_hbm, ov_hbm, oi_hbm, buf0, buf1, buft, res_f, res_i,
              sem0, sem1, sem2):
    cid = lax.axis_index("c")
    sid = lax.axis_index("s")
    wid = cid * 16 + sid
    v0 = pl.multiple_of(SC_V0 + wid * SC_STRIPE, 8)
    bufs = (buf0, buf1)
    sems = (sem0, sem1)

    def chunk_src(j):
        return x_hbm.at[pl.ds(pl.multiple_of(v0 + j * SC_CW, 8), SC_CW), :]

    srct = x_hbm.at[pl.ds(SC_TAIL0, SC_WTAIL), :]
    pltpu.async_copy(chunk_src(0), buf0, sem0)
    pltpu.async_copy(chunk_src(1), buf1, sem1)
    pltpu.async_copy(srct, buft, sem2)

    carry = (tuple(jnp.full((16,), -jnp.inf, jnp.float32) for _ in range(8)),
             tuple(jnp.zeros((16,), jnp.int32) for _ in range(8)))

    def scan(bufref, n, vbase, carry):
        def body(s, c):
            cvm = list(c[0])
            cvi = list(c[1])
            idx = jnp.full((16,), vbase + s, jnp.int32)
            for g in range(8):
                v = bufref[s, pl.ds(g * 16, 16)]
                pred = v > cvm[g]
                cvm[g] = jnp.where(pred, v, cvm[g])
                cvi[g] = jnp.where(pred, idx, cvi[g])
            return tuple(cvm), tuple(cvi)

        return lax.fori_loop(0, n, body, carry)

    for j in range(SC_NCH):
        b = j % 2
        pltpu.make_async_copy(chunk_src(j), bufs[b], sems[b]).wait()
        carry = scan(bufs[b], SC_CW, v0 + j * SC_CW, carry)
        if j + 2 < SC_NCH:
            pltpu.async_copy(chunk_src(j + 2), bufs[b], sems[b])

    pltpu.make_async_copy(srct, buft, sem2).wait()
    vm, vi = scan(buft, SC_WTAIL, jnp.int32(SC_TAIL0), carry)

    for g in range(8):
        res_f[pl.ds(g * 16, 16)] = vm[g]
        res_i[pl.ds(g * 16, 16)] = vi[g]
    pltpu.sync_copy(res_f, ov_hbm.at[wid])
    pltpu.sync_copy(res_i, oi_hbm.at[wid])

def _merge_body(tv_ref, ti_ref, sv_ref, si_ref, o_ref):
    sv = sv_ref[...]
    m = jnp.max(sv, axis=0, keepdims=True)
    cand = jnp.where(sv == m, si_ref[...], jnp.int32(IBIG))
    smi = jnp.min(cand, axis=0, keepdims=True)
    better = m > tv_ref[...]
    o_ref[...] = jnp.where(better, smi, ti_ref[...])


_merge = pl.pallas_call(
    _merge_body,
    out_shape=jax.ShapeDtypeStruct((1, R), jnp.int32),
)

@jax.jit
def kernel(m_logits):
    x_t = m_logits.T
    tv, ti = _tc_argmax(x_t)
    sv, si = _sc_shard(x_t)
    token = _merge(tv, ti, sv, si)
    return token.reshape(R, 1).astype(jnp.int64)
